# Initial kernel scaffold; baseline (speedup 1.0000x reference)
import jax
import jax.numpy as jnp
from jax.experimental import pallas as pl


def kernel(x, edge_index, edge_attr, We_w, We_b, Wn_w, Wn_b):
    raise NotImplementedError("write your pallas kernel here")


# speedup vs baseline: 5.4774x; 5.4774x over previous
import functools

import jax
import jax.numpy as jnp
from jax import lax
from jax.experimental import pallas as pl
from jax.experimental.pallas import tpu as pltpu
from jax.experimental.pallas import tpu_sc as plsc

F32 = jnp.float32

_NC = 2
_NS = 16
_SUB = 125
_CHUNK = 1000


def _proj_body(x_ref, we1_ref, we2_ref, wn1_ref, ps_ref, pd_ref, xw_ref):
    xb = x_ref[...]
    ps_ref[...] = jnp.dot(xb, we1_ref[...], preferred_element_type=F32)
    pd_ref[...] = jnp.dot(xb, we2_ref[...], preferred_element_type=F32)
    xw_ref[...] = jnp.dot(xb, wn1_ref[...], preferred_element_type=F32)


def _edge_lin_body(e_ref, m_ref, b_ref, t_ref):
    t_ref[...] = jnp.dot(e_ref[...], m_ref[...],
                         preferred_element_type=F32) + b_ref[...]


def _node_body(x_ref, xw_ref, acc_ref, cnt_ref, wn2_ref, wnb_ref, out_ref):
    agg_sum = acc_ref[0] + acc_ref[1]
    cnt = cnt_ref[0] + cnt_ref[1]
    agg = agg_sum / jnp.maximum(cnt, 1.0)
    out_ref[...] = (x_ref[...] + xw_ref[...] +
                    jnp.dot(agg, wn2_ref[...], preferred_element_type=F32) +
                    wnb_ref[...])


def _make_sc_kernel(E, N, DE, interpret=False):
    NW = _NC * _NS
    epw = E // NW
    n_chunks = epw // _CHUNK
    rows_per_chunk = _CHUNK // _SUB
    idx_rows_per_worker = epw // _SUB
    mesh = plsc.VectorSubcoreMesh(core_axis_name="c", subcore_axis_name="s",
                                  num_cores=_NC, num_subcores=_NS)

    @functools.partial(
        pl.kernel, mesh=mesh, interpret=interpret,
        compiler_params=pltpu.CompilerParams(use_tc_tiling_on_sc=False),
        out_type=(jax.ShapeDtypeStruct((E, DE), F32),
                  jax.ShapeDtypeStruct((_NC, N, DE), F32),
                  jax.ShapeDtypeStruct((_NC, N, DE), F32)),
        scratch_types=[
            pltpu.VMEM((rows_per_chunk, _SUB), jnp.int32),
            pltpu.VMEM((rows_per_chunk, _SUB), jnp.int32),
            pltpu.VMEM((_CHUNK, DE), F32),
            pltpu.VMEM((_CHUNK, DE), F32),
            pltpu.VMEM((_CHUNK, DE), F32),
            pltpu.VMEM((_CHUNK, DE), F32),
            pltpu.VMEM((_SUB, DE), F32),
            pltpu.VMEM_SHARED((N, DE), F32),
            pltpu.VMEM_SHARED((N, DE), F32),
            pltpu.SemaphoreType.DMA,
            pltpu.SemaphoreType.DMA,
        ])
    def sck(src_hbm, dst_hbm, t_hbm, ps_hbm, pd_hbm, zeros_hbm, ones_hbm,
            eout_hbm, acc_hbm, cnt_hbm,
            src_v, dst_v, t_v, pse_v, pde_v, enew_v, ones_v,
            acc_sh, cnt_sh, sem1, sem2):
        cid = lax.axis_index("c")
        sid = lax.axis_index("s")
        wid = sid * _NC + cid

        @pl.when(sid == 0)
        def _():
            pltpu.sync_copy(zeros_hbm, acc_sh)
            pltpu.sync_copy(zeros_hbm, cnt_sh)

        pltpu.sync_copy(ones_hbm, ones_v)
        plsc.subcore_barrier()

        def chunk(k, carry):
            base = wid * epw + k * _CHUNK
            idx_row = wid * idx_rows_per_worker + k * rows_per_chunk
            pltpu.sync_copy(src_hbm.at[pl.ds(idx_row, rows_per_chunk)], src_v)
            pltpu.sync_copy(dst_hbm.at[pl.ds(idx_row, rows_per_chunk)], dst_v)
            gathers = []
            for j in range(rows_per_chunk):
                sl = pl.ds(j * _SUB, _SUB)
                gathers.append(pltpu.async_copy(
                    ps_hbm.at[src_v.at[j]], pse_v.at[sl], sem1))
                gathers.append(pltpu.async_copy(
                    pd_hbm.at[dst_v.at[j]], pde_v.at[sl], sem2))
            pltpu.sync_copy(t_hbm.at[pl.ds(base, _CHUNK)], t_v)
            for g in gathers:
                g.wait()

            def row(i, c2):
                r = i * 4
                enew_v[r] = t_v[r] + pse_v[r] + pde_v[r]
                enew_v[r + 1] = t_v[r + 1] + pse_v[r + 1] + pde_v[r + 1]
                enew_v[r + 2] = t_v[r + 2] + pse_v[r + 2] + pde_v[r + 2]
                enew_v[r + 3] = t_v[r + 3] + pse_v[r + 3] + pde_v[r + 3]
                return c2

            lax.fori_loop(0, _CHUNK // 4, row, 0)
            st = pltpu.async_copy(enew_v, eout_hbm.at[pl.ds(base, _CHUNK)],
                                  sem1)
            scatters = []
            for j in range(rows_per_chunk):
                sl = pl.ds(j * _SUB, _SUB)
                scatters.append(pltpu.async_copy(
                    enew_v.at[sl], acc_sh.at[dst_v.at[j]], sem2, add=True))
                scatters.append(pltpu.async_copy(
                    ones_v, cnt_sh.at[dst_v.at[j]], sem2, add=True))
            st.wait()
            for s in scatters:
                s.wait()
            return carry

        lax.fori_loop(0, n_chunks, chunk, 0)
        plsc.subcore_barrier()

        @pl.when(sid == 0)
        def _():
            pltpu.sync_copy(acc_sh, acc_hbm.at[cid])
            pltpu.sync_copy(cnt_sh, cnt_hbm.at[cid])

    return sck


def kernel(x, edge_index, edge_attr, We_w, We_b, Wn_w, Wn_b):
    N, D = x.shape
    E, DE = edge_attr.shape
    src = edge_index[0]
    dst = edge_index[1]
    We1 = We_w[:D]
    We2 = We_w[D:2 * D]
    We3 = We_w[2 * D:]
    M = jnp.eye(DE, dtype=F32) + We3
    Wn1 = Wn_w[:D]
    Wn2 = Wn_w[D:]

    BN = 2000
    ps, pd_, xw = pl.pallas_call(
        _proj_body,
        grid=(N // BN,),
        in_specs=[pl.BlockSpec((BN, D), lambda i: (i, 0)),
                  pl.BlockSpec((D, DE), lambda i: (0, 0)),
                  pl.BlockSpec((D, DE), lambda i: (0, 0)),
                  pl.BlockSpec((D, D), lambda i: (0, 0))],
        out_specs=[pl.BlockSpec((BN, DE), lambda i: (i, 0)),
                   pl.BlockSpec((BN, DE), lambda i: (i, 0)),
                   pl.BlockSpec((BN, D), lambda i: (i, 0))],
        out_shape=[jax.ShapeDtypeStruct((N, DE), F32),
                   jax.ShapeDtypeStruct((N, DE), F32),
                   jax.ShapeDtypeStruct((N, D), F32)],
    )(x, We1, We2, Wn1)

    BE = 8000
    t = pl.pallas_call(
        _edge_lin_body,
        grid=(E // BE,),
        in_specs=[pl.BlockSpec((BE, DE), lambda i: (i, 0)),
                  pl.BlockSpec((DE, DE), lambda i: (0, 0)),
                  pl.BlockSpec((1, DE), lambda i: (0, 0))],
        out_specs=pl.BlockSpec((BE, DE), lambda i: (i, 0)),
        out_shape=jax.ShapeDtypeStruct((E, DE), F32),
    )(edge_attr, M, We_b.reshape(1, DE))

    src2 = src.reshape(E // _SUB, _SUB)
    dst2 = dst.reshape(E // _SUB, _SUB)
    zeros = jnp.zeros((N, DE), F32)
    ones = jnp.ones((_SUB, DE), F32)
    sck = _make_sc_kernel(E, N, DE)
    eout, acc, cnt = sck(src2, dst2, t, ps, pd_, zeros, ones)

    out_x = pl.pallas_call(
        _node_body,
        grid=(N // BN,),
        in_specs=[pl.BlockSpec((BN, D), lambda i: (i, 0)),
                  pl.BlockSpec((BN, D), lambda i: (i, 0)),
                  pl.BlockSpec((_NC, BN, DE), lambda i: (0, i, 0)),
                  pl.BlockSpec((_NC, BN, DE), lambda i: (0, i, 0)),
                  pl.BlockSpec((DE, D), lambda i: (0, 0)),
                  pl.BlockSpec((1, D), lambda i: (0, 0))],
        out_specs=pl.BlockSpec((BN, D), lambda i: (i, 0)),
        out_shape=jax.ShapeDtypeStruct((N, D), F32),
    )(x, xw, acc, cnt, Wn2, Wn_b.reshape(1, D))

    return (out_x, eout)


# speedup vs baseline: 6.4469x; 1.1770x over previous
import functools

import jax
import jax.numpy as jnp
from jax import lax
from jax.experimental import pallas as pl
from jax.experimental.pallas import tpu as pltpu
from jax.experimental.pallas import tpu_sc as plsc

F32 = jnp.float32

_NC = 2
_NS = 16
_SUB = 125
_CHUNK = 1000


def _proj_body(x_ref, we1_ref, we2_ref, wn1_ref, ps_ref, pd_ref, xw_ref):
    xb = x_ref[...]
    ps_ref[...] = jnp.dot(xb, we1_ref[...], preferred_element_type=F32)
    pd_ref[...] = jnp.dot(xb, we2_ref[...], preferred_element_type=F32)
    xw_ref[...] = jnp.dot(xb, wn1_ref[...], preferred_element_type=F32)


def _edge_lin_body(et_ref, m_ref, b_ref, t_ref):
    t_ref[...] = lax.dot_general(
        et_ref[...], m_ref[...],
        dimension_numbers=(((0,), (0,)), ((), ())),
        preferred_element_type=F32) + b_ref[...]


def _transpose_body(e_ref, out_ref):
    out_ref[...] = e_ref[...].T


def _node_body(x_ref, xw_ref, acc_ref, cnt_ref, wn2_ref, wnb_ref, out_ref):
    agg_sum = acc_ref[0] + acc_ref[1]
    cnt = cnt_ref[0] + cnt_ref[1]
    agg = agg_sum / jnp.maximum(cnt, 1.0)
    out_ref[...] = (x_ref[...] + xw_ref[...] +
                    jnp.dot(agg, wn2_ref[...], preferred_element_type=F32) +
                    wnb_ref[...])


def _make_sc_kernel(E, N, DE, interpret=False):
    NW = _NC * _NS
    epw = E // NW
    n_chunks = epw // _CHUNK
    rows_per_chunk = _CHUNK // _SUB
    idx_rows_per_worker = epw // _SUB
    mesh = plsc.VectorSubcoreMesh(core_axis_name="c", subcore_axis_name="s",
                                  num_cores=_NC, num_subcores=_NS)

    @functools.partial(
        pl.kernel, mesh=mesh, interpret=interpret,
        compiler_params=pltpu.CompilerParams(use_tc_tiling_on_sc=False),
        out_type=(jax.ShapeDtypeStruct((E, DE), F32),
                  jax.ShapeDtypeStruct((_NC, N, DE), F32),
                  jax.ShapeDtypeStruct((_NC, N, DE), F32)),
        scratch_types=[
            pltpu.VMEM((rows_per_chunk, _SUB), jnp.int32),
            pltpu.VMEM((rows_per_chunk, _SUB), jnp.int32),
            pltpu.VMEM((_CHUNK, DE), F32),
            pltpu.VMEM((_CHUNK, DE), F32),
            pltpu.VMEM((_CHUNK, DE), F32),
            pltpu.VMEM((_CHUNK, DE), F32),
            pltpu.VMEM((_SUB, DE), F32),
            pltpu.VMEM_SHARED((N, DE), F32),
            pltpu.VMEM_SHARED((N, DE), F32),
            pltpu.SemaphoreType.DMA,
            pltpu.SemaphoreType.DMA,
        ])
    def sck(src_hbm, dst_hbm, t_hbm, ps_hbm, pd_hbm, zeros_hbm, ones_hbm,
            eout_hbm, acc_hbm, cnt_hbm,
            src_v, dst_v, t_v, pse_v, pde_v, enew_v, ones_v,
            acc_sh, cnt_sh, sem1, sem2):
        cid = lax.axis_index("c")
        sid = lax.axis_index("s")
        wid = sid * _NC + cid

        @pl.when(sid == 0)
        def _():
            pltpu.sync_copy(zeros_hbm, acc_sh)
            pltpu.sync_copy(zeros_hbm, cnt_sh)

        pltpu.sync_copy(ones_hbm, ones_v)
        plsc.subcore_barrier()

        def chunk(k, carry):
            base = wid * epw + k * _CHUNK
            idx_row = wid * idx_rows_per_worker + k * rows_per_chunk
            pltpu.sync_copy(src_hbm.at[pl.ds(idx_row, rows_per_chunk)], src_v)
            pltpu.sync_copy(dst_hbm.at[pl.ds(idx_row, rows_per_chunk)], dst_v)
            gathers = []
            for j in range(rows_per_chunk):
                sl = pl.ds(j * _SUB, _SUB)
                gathers.append(pltpu.async_copy(
                    ps_hbm.at[src_v.at[j]], pse_v.at[sl], sem1))
                gathers.append(pltpu.async_copy(
                    pd_hbm.at[dst_v.at[j]], pde_v.at[sl], sem2))
            pltpu.sync_copy(t_hbm.at[pl.ds(base, _CHUNK)], t_v)
            for g in gathers:
                g.wait()

            def row(i, c2):
                r = i * 4
                enew_v[r] = t_v[r] + pse_v[r] + pde_v[r]
                enew_v[r + 1] = t_v[r + 1] + pse_v[r + 1] + pde_v[r + 1]
                enew_v[r + 2] = t_v[r + 2] + pse_v[r + 2] + pde_v[r + 2]
                enew_v[r + 3] = t_v[r + 3] + pse_v[r + 3] + pde_v[r + 3]
                return c2

            lax.fori_loop(0, _CHUNK // 4, row, 0)
            st = pltpu.async_copy(enew_v, eout_hbm.at[pl.ds(base, _CHUNK)],
                                  sem1)
            scatters = []
            for j in range(rows_per_chunk):
                sl = pl.ds(j * _SUB, _SUB)
                scatters.append(pltpu.async_copy(
                    enew_v.at[sl], acc_sh.at[dst_v.at[j]], sem2, add=True))
                scatters.append(pltpu.async_copy(
                    ones_v, cnt_sh.at[dst_v.at[j]], sem2, add=True))
            st.wait()
            for s in scatters:
                s.wait()
            return carry

        lax.fori_loop(0, n_chunks, chunk, 0)
        plsc.subcore_barrier()

        @pl.when(sid == 0)
        def _():
            pltpu.sync_copy(acc_sh, acc_hbm.at[cid])
            pltpu.sync_copy(cnt_sh, cnt_hbm.at[cid])

    return sck


def kernel(x, edge_index, edge_attr, We_w, We_b, Wn_w, Wn_b):
    N, D = x.shape
    E, DE = edge_attr.shape
    src = edge_index[0]
    dst = edge_index[1]
    We1 = We_w[:D]
    We2 = We_w[D:2 * D]
    We3 = We_w[2 * D:]
    M = jnp.eye(DE, dtype=F32) + We3
    Wn1 = Wn_w[:D]
    Wn2 = Wn_w[D:]

    BN = 2000
    ps, pd_, xw = pl.pallas_call(
        _proj_body,
        grid=(N // BN,),
        in_specs=[pl.BlockSpec((BN, D), lambda i: (i, 0)),
                  pl.BlockSpec((D, DE), lambda i: (0, 0)),
                  pl.BlockSpec((D, DE), lambda i: (0, 0)),
                  pl.BlockSpec((D, D), lambda i: (0, 0))],
        out_specs=[pl.BlockSpec((BN, DE), lambda i: (i, 0)),
                   pl.BlockSpec((BN, DE), lambda i: (i, 0)),
                   pl.BlockSpec((BN, D), lambda i: (i, 0))],
        out_shape=[jax.ShapeDtypeStruct((N, DE), F32),
                   jax.ShapeDtypeStruct((N, DE), F32),
                   jax.ShapeDtypeStruct((N, D), F32)],
    )(x, We1, We2, Wn1)

    BE = 12800
    ea_t = edge_attr.T
    t = pl.pallas_call(
        _edge_lin_body,
        grid=(E // BE,),
        in_specs=[pl.BlockSpec((DE, BE), lambda i: (0, i)),
                  pl.BlockSpec((DE, DE), lambda i: (0, 0)),
                  pl.BlockSpec((1, DE), lambda i: (0, 0))],
        out_specs=pl.BlockSpec((BE, DE), lambda i: (i, 0)),
        out_shape=jax.ShapeDtypeStruct((E, DE), F32),
    )(ea_t, M, We_b.reshape(1, DE))

    src2 = src.reshape(E // _SUB, _SUB)
    dst2 = dst.reshape(E // _SUB, _SUB)
    zeros = jnp.zeros((N, DE), F32)
    ones = jnp.ones((_SUB, DE), F32)
    sck = _make_sc_kernel(E, N, DE)
    eout, acc, cnt = sck(src2, dst2, t, ps, pd_, zeros, ones)

    out_x = pl.pallas_call(
        _node_body,
        grid=(N // BN,),
        in_specs=[pl.BlockSpec((BN, D), lambda i: (i, 0)),
                  pl.BlockSpec((BN, D), lambda i: (i, 0)),
                  pl.BlockSpec((_NC, BN, DE), lambda i: (0, i, 0)),
                  pl.BlockSpec((_NC, BN, DE), lambda i: (0, i, 0)),
                  pl.BlockSpec((DE, D), lambda i: (0, 0)),
                  pl.BlockSpec((1, D), lambda i: (0, 0))],
        out_specs=pl.BlockSpec((BN, D), lambda i: (i, 0)),
        out_shape=jax.ShapeDtypeStruct((N, D), F32),
    )(x, xw, acc, cnt, Wn2, Wn_b.reshape(1, D))

    BT = 12800
    eout_t = pl.pallas_call(
        _transpose_body,
        grid=(E // BT,),
        in_specs=[pl.BlockSpec((BT, DE), lambda i: (i, 0))],
        out_specs=pl.BlockSpec((DE, BT), lambda i: (0, i)),
        out_shape=jax.ShapeDtypeStruct((DE, E), F32),
    )(eout)

    return (out_x, eout_t.T)


# speedup vs baseline: 10.9072x; 1.6919x over previous
import functools

import jax
import jax.numpy as jnp
from jax import lax
from jax.experimental import pallas as pl
from jax.experimental.pallas import tpu as pltpu
from jax.experimental.pallas import tpu_sc as plsc

F32 = jnp.float32

_NC = 2
_NS = 16
_SUB = 125
_CHUNK = 1000


def _proj_body(x_ref, we1_ref, we2_ref, wn1_ref, ps_ref, pd_ref, xw_ref):
    xb = x_ref[...]
    ps_ref[...] = jnp.dot(xb, we1_ref[...], preferred_element_type=F32)
    pd_ref[...] = jnp.dot(xb, we2_ref[...], preferred_element_type=F32)
    xw_ref[...] = jnp.dot(xb, wn1_ref[...], preferred_element_type=F32)


def _edge_lin_body(et_ref, m_ref, b_ref, t_ref):
    t_ref[...] = lax.dot_general(
        m_ref[...], et_ref[...],
        dimension_numbers=(((0,), (0,)), ((), ())),
        preferred_element_type=F32) + b_ref[...]


def _node_body(x_ref, xw_ref, acc_ref, cnt_ref, wn2_ref, wnb_ref, out_ref):
    agg_sum = acc_ref[0] + acc_ref[1]
    cnt = cnt_ref[0] + cnt_ref[1]
    agg = agg_sum / jnp.maximum(cnt, 1.0)
    out_ref[...] = (x_ref[...] + xw_ref[...] +
                    jnp.dot(agg, wn2_ref[...], preferred_element_type=F32) +
                    wnb_ref[...])


def _make_sc_kernel(E, N, DE, interpret=False):
    NW = _NC * _NS
    epw = E // NW
    n_chunks = epw // _CHUNK
    rows_per_chunk = _CHUNK // _SUB
    idx_rows_per_worker = epw // _SUB
    mesh = plsc.VectorSubcoreMesh(core_axis_name="c", subcore_axis_name="s",
                                  num_cores=_NC, num_subcores=_NS)

    @functools.partial(
        pl.kernel, mesh=mesh, interpret=interpret,
        compiler_params=pltpu.CompilerParams(use_tc_tiling_on_sc=False,
                                             needs_layout_passes=False),
        out_type=(jax.ShapeDtypeStruct((DE, E), F32),
                  jax.ShapeDtypeStruct((_NC, N, DE), F32),
                  jax.ShapeDtypeStruct((_NC, N, DE), F32)),
        scratch_types=[
            pltpu.VMEM((rows_per_chunk, _SUB), jnp.int32),
            pltpu.VMEM((rows_per_chunk, _SUB), jnp.int32),
            pltpu.VMEM((DE * _CHUNK,), F32),
            pltpu.VMEM((_CHUNK, DE), F32),
            pltpu.VMEM((_CHUNK, DE), F32),
            pltpu.VMEM((_CHUNK, DE), F32),
            pltpu.VMEM((DE * _CHUNK,), F32),
            pltpu.VMEM((_SUB, DE), F32),
            pltpu.VMEM_SHARED((N, DE), F32),
            pltpu.VMEM_SHARED((N, DE), F32),
            pltpu.SemaphoreType.DMA,
            pltpu.SemaphoreType.DMA,
        ])
    def sck(ei_hbm, t_hbm, ps_hbm, pd_hbm, zeros_hbm, ones_hbm,
            eout_hbm, acc_hbm, cnt_hbm,
            src_v, dst_v, tc_v, pse_v, pde_v, enew_v, eoc_v, ones_v,
            acc_sh, cnt_sh, sem1, sem2):
        cid = lax.axis_index("c")
        sid = lax.axis_index("s")
        wid = sid * _NC + cid

        @pl.when(sid == 0)
        def _():
            pltpu.sync_copy(zeros_hbm, acc_sh)
            pltpu.sync_copy(zeros_hbm, cnt_sh)

        pltpu.sync_copy(ones_hbm, ones_v)
        plsc.subcore_barrier()
        lane_base = lax.iota(jnp.int32, DE) * _CHUNK

        def chunk(k, carry):
            base = wid * epw + k * _CHUNK
            idx_row = wid * idx_rows_per_worker + k * rows_per_chunk
            pltpu.sync_copy(ei_hbm.at[0, pl.ds(idx_row, rows_per_chunk)], src_v)
            pltpu.sync_copy(ei_hbm.at[1, pl.ds(idx_row, rows_per_chunk)], dst_v)
            gathers = []
            for j in range(rows_per_chunk):
                sl = pl.ds(j * _SUB, _SUB)
                gathers.append(pltpu.async_copy(
                    ps_hbm.at[src_v.at[j]], pse_v.at[sl], sem1))
                gathers.append(pltpu.async_copy(
                    pd_hbm.at[dst_v.at[j]], pde_v.at[sl], sem2))
            for f in range(DE):
                gathers.append(pltpu.async_copy(
                    t_hbm.at[f, pl.ds(base, _CHUNK)],
                    tc_v.at[pl.ds(f * _CHUNK, _CHUNK)], sem1))
            for g in gathers:
                g.wait()

            def row(i, c2):
                r = i * 4
                for u in range(4):
                    idx = lane_base + (r + u)
                    tr = plsc.load_gather(tc_v, [idx])
                    enew = tr + pse_v[r + u] + pde_v[r + u]
                    enew_v[r + u] = enew
                    plsc.store_scatter(eoc_v, [idx], enew)
                return c2

            lax.fori_loop(0, _CHUNK // 4, row, 0)
            stores = []
            for f in range(DE):
                stores.append(pltpu.async_copy(
                    eoc_v.at[pl.ds(f * _CHUNK, _CHUNK)],
                    eout_hbm.at[f, pl.ds(base, _CHUNK)], sem1))
            for j in range(rows_per_chunk):
                sl = pl.ds(j * _SUB, _SUB)
                stores.append(pltpu.async_copy(
                    enew_v.at[sl], acc_sh.at[dst_v.at[j]], sem2, add=True))
                stores.append(pltpu.async_copy(
                    ones_v, cnt_sh.at[dst_v.at[j]], sem2, add=True))
            for s in stores:
                s.wait()
            return carry

        lax.fori_loop(0, n_chunks, chunk, 0)
        plsc.subcore_barrier()

        @pl.when(sid == 0)
        def _():
            pltpu.sync_copy(acc_sh, acc_hbm.at[cid])
            pltpu.sync_copy(cnt_sh, cnt_hbm.at[cid])

    return sck


def kernel(x, edge_index, edge_attr, We_w, We_b, Wn_w, Wn_b):
    N, D = x.shape
    E, DE = edge_attr.shape
    src = edge_index[0]
    dst = edge_index[1]
    We1 = We_w[:D]
    We2 = We_w[D:2 * D]
    We3 = We_w[2 * D:]
    M = jnp.eye(DE, dtype=F32) + We3
    Wn1 = Wn_w[:D]
    Wn2 = Wn_w[D:]

    BN = 2000
    ps, pd_, xw = pl.pallas_call(
        _proj_body,
        grid=(N // BN,),
        in_specs=[pl.BlockSpec((BN, D), lambda i: (i, 0)),
                  pl.BlockSpec((D, DE), lambda i: (0, 0)),
                  pl.BlockSpec((D, DE), lambda i: (0, 0)),
                  pl.BlockSpec((D, D), lambda i: (0, 0))],
        out_specs=[pl.BlockSpec((BN, DE), lambda i: (i, 0)),
                   pl.BlockSpec((BN, DE), lambda i: (i, 0)),
                   pl.BlockSpec((BN, D), lambda i: (i, 0))],
        out_shape=[jax.ShapeDtypeStruct((N, DE), F32),
                   jax.ShapeDtypeStruct((N, DE), F32),
                   jax.ShapeDtypeStruct((N, D), F32)],
    )(x, We1, We2, Wn1)

    BE = 12800
    ea_t = edge_attr.T
    t_t = pl.pallas_call(
        _edge_lin_body,
        grid=(E // BE,),
        in_specs=[pl.BlockSpec((DE, BE), lambda i: (0, i)),
                  pl.BlockSpec((DE, DE), lambda i: (0, 0)),
                  pl.BlockSpec((DE, 1), lambda i: (0, 0))],
        out_specs=pl.BlockSpec((DE, BE), lambda i: (0, i)),
        out_shape=jax.ShapeDtypeStruct((DE, E), F32),
    )(ea_t, M, We_b.reshape(DE, 1))

    ei3 = edge_index.reshape(2, E // _SUB, _SUB)
    zeros = jnp.zeros((N, DE), F32)
    ones = jnp.ones((_SUB, DE), F32)
    sck = _make_sc_kernel(E, N, DE)
    eout_t, acc, cnt = sck(ei3, t_t, ps, pd_, zeros, ones)

    out_x = pl.pallas_call(
        _node_body,
        grid=(N // BN,),
        in_specs=[pl.BlockSpec((BN, D), lambda i: (i, 0)),
                  pl.BlockSpec((BN, D), lambda i: (i, 0)),
                  pl.BlockSpec((_NC, BN, DE), lambda i: (0, i, 0)),
                  pl.BlockSpec((_NC, BN, DE), lambda i: (0, i, 0)),
                  pl.BlockSpec((DE, D), lambda i: (0, 0)),
                  pl.BlockSpec((1, D), lambda i: (0, 0))],
        out_specs=pl.BlockSpec((BN, D), lambda i: (i, 0)),
        out_shape=jax.ShapeDtypeStruct((N, D), F32),
    )(x, xw, acc, cnt, Wn2, Wn_b.reshape(1, D))

    return (out_x, eout_t.T)


# speedup vs baseline: 14.2306x; 1.3047x over previous
import functools

import jax
import jax.numpy as jnp
from jax import lax
from jax.experimental import pallas as pl
from jax.experimental.pallas import tpu as pltpu
from jax.experimental.pallas import tpu_sc as plsc

F32 = jnp.float32

_NC = 2
_NS = 16
_SUB = 125
_CHUNK = 1000


def _proj_body(x_ref, we1_ref, we2_ref, wn1_ref, ps_ref, pd_ref, xw_ref):
    xb = x_ref[...]
    ps_ref[...] = jnp.dot(xb, we1_ref[...], preferred_element_type=F32)
    pd_ref[...] = jnp.dot(xb, we2_ref[...], preferred_element_type=F32)
    xw_ref[...] = jnp.dot(xb, wn1_ref[...], preferred_element_type=F32)


def _edge_lin_body(et_ref, m_ref, b_ref, t_ref):
    t_ref[...] = lax.dot_general(
        m_ref[...], et_ref[...],
        dimension_numbers=(((0,), (0,)), ((), ())),
        preferred_element_type=F32) + b_ref[...]


def _node_body(x_ref, xw_ref, acc_ref, cnt_ref, wn2_ref, wnb_ref, out_ref):
    agg_sum = acc_ref[0] + acc_ref[1]
    cnt = cnt_ref[0] + cnt_ref[1]
    agg = agg_sum / jnp.maximum(cnt, 1.0)
    out_ref[...] = (x_ref[...] + xw_ref[...] +
                    jnp.dot(agg, wn2_ref[...], preferred_element_type=F32) +
                    wnb_ref[...])


def _make_sc_kernel(E, N, DE, interpret=False):
    NW = _NC * _NS
    epw = E // NW
    n_chunks = epw // _CHUNK
    rows_per_chunk = _CHUNK // _SUB
    idx_rows_per_worker = epw // _SUB
    mesh = plsc.VectorSubcoreMesh(core_axis_name="c", subcore_axis_name="s",
                                  num_cores=_NC, num_subcores=_NS)

    @functools.partial(
        pl.kernel, mesh=mesh, interpret=interpret,
        compiler_params=pltpu.CompilerParams(use_tc_tiling_on_sc=False,
                                             needs_layout_passes=False),
        out_type=(jax.ShapeDtypeStruct((DE, E), F32),
                  jax.ShapeDtypeStruct((_NC, N, DE), F32),
                  jax.ShapeDtypeStruct((_NC, N, DE), F32)),
        scratch_types=[
            pltpu.VMEM((rows_per_chunk, _SUB), jnp.int32),
            pltpu.VMEM((rows_per_chunk, _SUB), jnp.int32),
            pltpu.VMEM((DE * _CHUNK,), F32),
            pltpu.VMEM((_CHUNK, DE), F32),
            pltpu.VMEM((_CHUNK, DE), F32),
            pltpu.VMEM((_CHUNK, DE), F32),
            pltpu.VMEM((DE * _CHUNK,), F32),
            pltpu.VMEM((_SUB, DE), F32),
            pltpu.VMEM_SHARED((N, DE), F32),
            pltpu.VMEM_SHARED((N, DE), F32),
            pltpu.SemaphoreType.DMA,
            pltpu.SemaphoreType.DMA,
        ])
    def sck(ei_hbm, t_hbm, ps_hbm, pd_hbm, zeros_hbm, ones_hbm,
            eout_hbm, acc_hbm, cnt_hbm,
            src_v, dst_v, tc_v, pse_v, pde_v, enew_v, eoc_v, ones_v,
            acc_sh, cnt_sh, sem1, sem2):
        cid = lax.axis_index("c")
        sid = lax.axis_index("s")
        wid = sid * _NC + cid

        @pl.when(sid == 0)
        def _():
            pltpu.sync_copy(zeros_hbm, acc_sh)
            pltpu.sync_copy(zeros_hbm, cnt_sh)

        pltpu.sync_copy(ones_hbm, ones_v)
        plsc.subcore_barrier()
        lane_base = lax.iota(jnp.int32, DE) * _CHUNK

        def chunk(k, carry):
            base = wid * epw + k * _CHUNK
            idx_row = wid * idx_rows_per_worker + k * rows_per_chunk
            pltpu.sync_copy(ei_hbm.at[0, pl.ds(idx_row, rows_per_chunk)], src_v)
            pltpu.sync_copy(ei_hbm.at[1, pl.ds(idx_row, rows_per_chunk)], dst_v)
            gathers = []
            for j in range(rows_per_chunk):
                sl = pl.ds(j * _SUB, _SUB)
                gathers.append(pltpu.async_copy(
                    ps_hbm.at[src_v.at[j]], pse_v.at[sl], sem1))
                gathers.append(pltpu.async_copy(
                    pd_hbm.at[dst_v.at[j]], pde_v.at[sl], sem2))
            for f in range(DE):
                gathers.append(pltpu.async_copy(
                    t_hbm.at[f, pl.ds(base, _CHUNK)],
                    tc_v.at[pl.ds(f * _CHUNK, _CHUNK)], sem1))
            for g in gathers:
                g.wait()

            @plsc.parallel_loop(0, _CHUNK, step=1, unroll=8)
            def _row(e):
                idx = lane_base + e
                tr = plsc.load_gather(tc_v, [idx])
                enew = tr + pse_v[e] + pde_v[e]
                enew_v[e] = enew
                plsc.store_scatter(eoc_v, [idx], enew)
            stores = []
            for f in range(DE):
                stores.append(pltpu.async_copy(
                    eoc_v.at[pl.ds(f * _CHUNK, _CHUNK)],
                    eout_hbm.at[f, pl.ds(base, _CHUNK)], sem1))
            for j in range(rows_per_chunk):
                sl = pl.ds(j * _SUB, _SUB)
                stores.append(pltpu.async_copy(
                    enew_v.at[sl], acc_sh.at[dst_v.at[j]], sem2, add=True))
                stores.append(pltpu.async_copy(
                    ones_v, cnt_sh.at[dst_v.at[j]], sem2, add=True))
            for s in stores:
                s.wait()
            return carry

        lax.fori_loop(0, n_chunks, chunk, 0)
        plsc.subcore_barrier()

        @pl.when(sid == 0)
        def _():
            pltpu.sync_copy(acc_sh, acc_hbm.at[cid])
            pltpu.sync_copy(cnt_sh, cnt_hbm.at[cid])

    return sck


def kernel(x, edge_index, edge_attr, We_w, We_b, Wn_w, Wn_b):
    N, D = x.shape
    E, DE = edge_attr.shape
    src = edge_index[0]
    dst = edge_index[1]
    We1 = We_w[:D]
    We2 = We_w[D:2 * D]
    We3 = We_w[2 * D:]
    M = jnp.eye(DE, dtype=F32) + We3
    Wn1 = Wn_w[:D]
    Wn2 = Wn_w[D:]

    BN = 2000
    ps, pd_, xw = pl.pallas_call(
        _proj_body,
        grid=(N // BN,),
        in_specs=[pl.BlockSpec((BN, D), lambda i: (i, 0)),
                  pl.BlockSpec((D, DE), lambda i: (0, 0)),
                  pl.BlockSpec((D, DE), lambda i: (0, 0)),
                  pl.BlockSpec((D, D), lambda i: (0, 0))],
        out_specs=[pl.BlockSpec((BN, DE), lambda i: (i, 0)),
                   pl.BlockSpec((BN, DE), lambda i: (i, 0)),
                   pl.BlockSpec((BN, D), lambda i: (i, 0))],
        out_shape=[jax.ShapeDtypeStruct((N, DE), F32),
                   jax.ShapeDtypeStruct((N, DE), F32),
                   jax.ShapeDtypeStruct((N, D), F32)],
    )(x, We1, We2, Wn1)

    BE = 12800
    ea_t = edge_attr.T
    t_t = pl.pallas_call(
        _edge_lin_body,
        grid=(E // BE,),
        in_specs=[pl.BlockSpec((DE, BE), lambda i: (0, i)),
                  pl.BlockSpec((DE, DE), lambda i: (0, 0)),
                  pl.BlockSpec((DE, 1), lambda i: (0, 0))],
        out_specs=pl.BlockSpec((DE, BE), lambda i: (0, i)),
        out_shape=jax.ShapeDtypeStruct((DE, E), F32),
    )(ea_t, M, We_b.reshape(DE, 1))

    ei3 = edge_index.reshape(2, E // _SUB, _SUB)
    zeros = jnp.zeros((N, DE), F32)
    ones = jnp.ones((_SUB, DE), F32)
    sck = _make_sc_kernel(E, N, DE)
    eout_t, acc, cnt = sck(ei3, t_t, ps, pd_, zeros, ones)

    out_x = pl.pallas_call(
        _node_body,
        grid=(N // BN,),
        in_specs=[pl.BlockSpec((BN, D), lambda i: (i, 0)),
                  pl.BlockSpec((BN, D), lambda i: (i, 0)),
                  pl.BlockSpec((_NC, BN, DE), lambda i: (0, i, 0)),
                  pl.BlockSpec((_NC, BN, DE), lambda i: (0, i, 0)),
                  pl.BlockSpec((DE, D), lambda i: (0, 0)),
                  pl.BlockSpec((1, D), lambda i: (0, 0))],
        out_specs=pl.BlockSpec((BN, D), lambda i: (i, 0)),
        out_shape=jax.ShapeDtypeStruct((N, D), F32),
    )(x, xw, acc, cnt, Wn2, Wn_b.reshape(1, D))

    return (out_x, eout_t.T)


# speedup vs baseline: 14.2647x; 1.0024x over previous
import functools

import jax
import jax.numpy as jnp
from jax import lax
from jax.experimental import pallas as pl
from jax.experimental.pallas import tpu as pltpu
from jax.experimental.pallas import tpu_sc as plsc

F32 = jnp.float32

_NC = 2
_NS = 16
_SUB = 125
_CHUNK = 1000


def _proj_body(x_ref, we1_ref, we2_ref, wn1_ref, ps_ref, pd_ref, xw_ref):
    xb = x_ref[...]
    ps_ref[...] = jnp.dot(xb, we1_ref[...], preferred_element_type=F32)
    pd_ref[...] = jnp.dot(xb, we2_ref[...], preferred_element_type=F32)
    xw_ref[...] = jnp.dot(xb, wn1_ref[...], preferred_element_type=F32)


def _edge_lin_body(et_ref, m_ref, b_ref, t_ref):
    t_ref[...] = lax.dot_general(
        m_ref[...], et_ref[...],
        dimension_numbers=(((0,), (0,)), ((), ())),
        preferred_element_type=F32) + b_ref[...]


def _node_body(x_ref, xw_ref, acc_ref, cnt_ref, wn2_ref, wnb_ref, out_ref):
    agg_sum = acc_ref[0] + acc_ref[1]
    cnt = cnt_ref[0] + cnt_ref[1]
    agg = agg_sum / jnp.maximum(cnt, 1.0)
    out_ref[...] = (x_ref[...] + xw_ref[...] +
                    jnp.dot(agg, wn2_ref[...], preferred_element_type=F32) +
                    wnb_ref[...])


def _make_sc_kernel(E, N, DE, interpret=False):
    NW = _NC * _NS
    epw = E // NW
    n_chunks = epw // _CHUNK
    rows_per_chunk = _CHUNK // _SUB
    idx_rows_per_worker = epw // _SUB
    mesh = plsc.VectorSubcoreMesh(core_axis_name="c", subcore_axis_name="s",
                                  num_cores=_NC, num_subcores=_NS)

    @functools.partial(
        pl.kernel, mesh=mesh, interpret=interpret,
        compiler_params=pltpu.CompilerParams(use_tc_tiling_on_sc=False,
                                             needs_layout_passes=False),
        out_type=(jax.ShapeDtypeStruct((DE, E), F32),
                  jax.ShapeDtypeStruct((_NC, N, DE), F32),
                  jax.ShapeDtypeStruct((_NC, N, DE), F32)),
        scratch_types=[
            pltpu.VMEM((rows_per_chunk, _SUB), jnp.int32),
            pltpu.VMEM((rows_per_chunk, _SUB), jnp.int32),
            pltpu.VMEM((DE * _CHUNK,), F32),
            pltpu.VMEM((_CHUNK, DE), F32),
            pltpu.VMEM((_CHUNK, DE), F32),
            pltpu.VMEM((_CHUNK, DE), F32),
            pltpu.VMEM((DE * _CHUNK,), F32),
            pltpu.VMEM((_SUB, DE), F32),
            pltpu.VMEM_SHARED((N, DE), F32),
            pltpu.VMEM_SHARED((N, DE), F32),
            pltpu.SemaphoreType.DMA,
            pltpu.SemaphoreType.DMA,
        ])
    def sck(ei_hbm, t_hbm, ps_hbm, pd_hbm, zeros_hbm, ones_hbm,
            eout_hbm, acc_hbm, cnt_hbm,
            src_v, dst_v, tc_v, pse_v, pde_v, enew_v, eoc_v, ones_v,
            acc_sh, cnt_sh, sem1, sem2):
        cid = lax.axis_index("c")
        sid = lax.axis_index("s")
        wid = sid * _NC + cid

        @pl.when(sid == 0)
        def _():
            pltpu.sync_copy(zeros_hbm, acc_sh)
            pltpu.sync_copy(zeros_hbm, cnt_sh)

        pltpu.sync_copy(ones_hbm, ones_v)
        plsc.subcore_barrier()
        lane_base = lax.iota(jnp.int32, DE) * _CHUNK

        def chunk(k, carry):
            base = wid * epw + k * _CHUNK
            idx_row = wid * idx_rows_per_worker + k * rows_per_chunk
            pltpu.sync_copy(ei_hbm.at[0, pl.ds(idx_row, rows_per_chunk)], src_v)
            pltpu.sync_copy(ei_hbm.at[1, pl.ds(idx_row, rows_per_chunk)], dst_v)
            gathers = []
            for j in range(rows_per_chunk):
                sl = pl.ds(j * _SUB, _SUB)
                gathers.append(pltpu.async_copy(
                    ps_hbm.at[src_v.at[j]], pse_v.at[sl], sem1))
                gathers.append(pltpu.async_copy(
                    pd_hbm.at[dst_v.at[j]], pde_v.at[sl], sem2))
            for f in range(DE):
                gathers.append(pltpu.async_copy(
                    t_hbm.at[f, pl.ds(base, _CHUNK)],
                    tc_v.at[pl.ds(f * _CHUNK, _CHUNK)], sem1))
            for g in gathers:
                g.wait()

            @plsc.parallel_loop(0, _CHUNK, step=1, unroll=16)
            def _row(e):
                idx = lane_base + e
                tr = plsc.load_gather(tc_v, [idx])
                enew = tr + pse_v[e] + pde_v[e]
                enew_v[e] = enew
                plsc.store_scatter(eoc_v, [idx], enew)
            stores = []
            for f in range(DE):
                stores.append(pltpu.async_copy(
                    eoc_v.at[pl.ds(f * _CHUNK, _CHUNK)],
                    eout_hbm.at[f, pl.ds(base, _CHUNK)], sem1))
            for j in range(rows_per_chunk):
                sl = pl.ds(j * _SUB, _SUB)
                stores.append(pltpu.async_copy(
                    enew_v.at[sl], acc_sh.at[dst_v.at[j]], sem2, add=True))
                stores.append(pltpu.async_copy(
                    ones_v, cnt_sh.at[dst_v.at[j]], sem2, add=True))
            for s in stores:
                s.wait()
            return carry

        lax.fori_loop(0, n_chunks, chunk, 0)
        plsc.subcore_barrier()

        @pl.when(sid == 0)
        def _():
            pltpu.sync_copy(acc_sh, acc_hbm.at[cid])
            pltpu.sync_copy(cnt_sh, cnt_hbm.at[cid])

    return sck


def kernel(x, edge_index, edge_attr, We_w, We_b, Wn_w, Wn_b):
    N, D = x.shape
    E, DE = edge_attr.shape
    src = edge_index[0]
    dst = edge_index[1]
    We1 = We_w[:D]
    We2 = We_w[D:2 * D]
    We3 = We_w[2 * D:]
    M = jnp.eye(DE, dtype=F32) + We3
    Wn1 = Wn_w[:D]
    Wn2 = Wn_w[D:]

    BN = 2000
    ps, pd_, xw = pl.pallas_call(
        _proj_body,
        grid=(N // BN,),
        in_specs=[pl.BlockSpec((BN, D), lambda i: (i, 0)),
                  pl.BlockSpec((D, DE), lambda i: (0, 0)),
                  pl.BlockSpec((D, DE), lambda i: (0, 0)),
                  pl.BlockSpec((D, D), lambda i: (0, 0))],
        out_specs=[pl.BlockSpec((BN, DE), lambda i: (i, 0)),
                   pl.BlockSpec((BN, DE), lambda i: (i, 0)),
                   pl.BlockSpec((BN, D), lambda i: (i, 0))],
        out_shape=[jax.ShapeDtypeStruct((N, DE), F32),
                   jax.ShapeDtypeStruct((N, DE), F32),
                   jax.ShapeDtypeStruct((N, D), F32)],
    )(x, We1, We2, Wn1)

    BE = 12800
    ea_t = edge_attr.T
    t_t = pl.pallas_call(
        _edge_lin_body,
        grid=(E // BE,),
        in_specs=[pl.BlockSpec((DE, BE), lambda i: (0, i)),
                  pl.BlockSpec((DE, DE), lambda i: (0, 0)),
                  pl.BlockSpec((DE, 1), lambda i: (0, 0))],
        out_specs=pl.BlockSpec((DE, BE), lambda i: (0, i)),
        out_shape=jax.ShapeDtypeStruct((DE, E), F32),
    )(ea_t, M, We_b.reshape(DE, 1))

    ei3 = edge_index.reshape(2, E // _SUB, _SUB)
    zeros = jnp.zeros((N, DE), F32)
    ones = jnp.ones((_SUB, DE), F32)
    sck = _make_sc_kernel(E, N, DE)
    eout_t, acc, cnt = sck(ei3, t_t, ps, pd_, zeros, ones)

    out_x = pl.pallas_call(
        _node_body,
        grid=(N // BN,),
        in_specs=[pl.BlockSpec((BN, D), lambda i: (i, 0)),
                  pl.BlockSpec((BN, D), lambda i: (i, 0)),
                  pl.BlockSpec((_NC, BN, DE), lambda i: (0, i, 0)),
                  pl.BlockSpec((_NC, BN, DE), lambda i: (0, i, 0)),
                  pl.BlockSpec((DE, D), lambda i: (0, 0)),
                  pl.BlockSpec((1, D), lambda i: (0, 0))],
        out_specs=pl.BlockSpec((BN, D), lambda i: (i, 0)),
        out_shape=jax.ShapeDtypeStruct((N, D), F32),
    )(x, xw, acc, cnt, Wn2, Wn_b.reshape(1, D))

    return (out_x, eout_t.T)
